# baseline (speedup 1.0000x reference)
import jax
import jax.numpy as jnp
from jax.experimental import pallas as pl
from jax.experimental.pallas import tpu as pltpu

_NUM_LEVELS = 4
_N = 1024
_D = 256
_B = 16384
_R = 256
_C = 2
_RB = _R * _C


def _prep_body(cb_ref, c2_ref, cbm2_ref, cb1_ref, cb2_ref, cb3_ref):
    cb = cb_ref[...]
    c2_ref[...] = jnp.sum(cb * cb, axis=-1)
    cb1 = cb.astype(jnp.bfloat16)
    rem = cb - cb1.astype(jnp.float32)
    cb2 = rem.astype(jnp.bfloat16)
    cb3 = (rem - cb2.astype(jnp.float32)).astype(jnp.bfloat16)
    cbm2_ref[...] = jnp.bfloat16(-2.0) * cb1
    cb1_ref[...] = cb1
    cb2_ref[...] = cb2
    cb3_ref[...] = cb3


def _mm_nt(a, b):
    return jax.lax.dot_general(
        a, b, (((1,), (1,)), ((), ())), preferred_element_type=jnp.float32)


def _mm_nn(a, b):
    return jax.lax.dot_general(
        a, b, (((1,), (0,)), ((), ())), preferred_element_type=jnp.float32)


def _vq_body(z_ref, c2_ref, cbm2_ref, cb1_ref, cb2_ref, cb3_ref,
             zq_ref, codes_ref, loss_ref):
    pid = pl.program_id(0)
    iota = jax.lax.broadcasted_iota(jnp.int32, (_R, _N), 1)
    rs = [z_ref[pl.ds(c * _R, _R), :] for c in range(_C)]
    zs = list(rs)
    for lvl in range(_NUM_LEVELS):
        cbm2 = cbm2_ref[lvl]
        cb1 = cb1_ref[lvl]
        cb2 = cb2_ref[lvl]
        cb3 = cb3_ref[lvl]
        c2 = c2_ref[lvl][None, :]
        mms = [_mm_nt(rs[c].astype(jnp.bfloat16), cbm2) for c in range(_C)]
        for c in range(_C):
            r = rs[c]
            r2 = jnp.sum(r * r, axis=1, keepdims=True)
            d = r2 + mms[c] + c2
            dmin = jnp.min(d, axis=1, keepdims=True)
            code = jnp.min(jnp.where(d == dmin, iota, _N), axis=1)
            codes_ref[pl.ds(c * _R, _R), pl.ds(lvl, 1)] = code[:, None]
            onehot = (iota == code[:, None]).astype(jnp.bfloat16)
            sel = (_mm_nn(onehot, cb1) + _mm_nn(onehot, cb2)) + _mm_nn(onehot, cb3)
            rs[c] = r - sel
    part = jnp.zeros((1, 1), jnp.float32)
    for c in range(_C):
        zq_ref[pl.ds(c * _R, _R), :] = zs[c] - rs[c]
        part += jnp.sum(rs[c] * rs[c]).reshape(1, 1)

    @pl.when(pid == 0)
    def _init():
        loss_ref[...] = jnp.zeros_like(loss_ref)

    loss_ref[...] += part * (1.0 / (_B * _D))


@jax.jit
def kernel(z, codebooks):
    cbs_shape = jax.ShapeDtypeStruct((_NUM_LEVELS, _N, _D), jnp.bfloat16)
    c2, cbm2, cb1, cb2, cb3 = pl.pallas_call(
        _prep_body,
        out_shape=[
            jax.ShapeDtypeStruct((_NUM_LEVELS, _N), jnp.float32),
            cbs_shape, cbs_shape, cbs_shape, cbs_shape,
        ],
    )(codebooks)

    grid = _B // _RB
    zq, codes_t, loss = pl.pallas_call(
        _vq_body,
        grid=(grid,),
        in_specs=[
            pl.BlockSpec((_RB, _D), lambda i: (i, 0)),
            pl.BlockSpec((_NUM_LEVELS, _N), lambda i: (0, 0)),
            pl.BlockSpec((_NUM_LEVELS, _N, _D), lambda i: (0, 0, 0)),
            pl.BlockSpec((_NUM_LEVELS, _N, _D), lambda i: (0, 0, 0)),
            pl.BlockSpec((_NUM_LEVELS, _N, _D), lambda i: (0, 0, 0)),
            pl.BlockSpec((_NUM_LEVELS, _N, _D), lambda i: (0, 0, 0)),
        ],
        out_specs=[
            pl.BlockSpec((_RB, _D), lambda i: (i, 0)),
            pl.BlockSpec((_RB, _NUM_LEVELS), lambda i: (i, 0)),
            pl.BlockSpec((1, 1), lambda i: (0, 0)),
        ],
        out_shape=[
            jax.ShapeDtypeStruct((_B, _D), jnp.float32),
            jax.ShapeDtypeStruct((_B, _NUM_LEVELS), jnp.int32),
            jax.ShapeDtypeStruct((1, 1), jnp.float32),
        ],
        compiler_params=pltpu.CompilerParams(
            dimension_semantics=("arbitrary",),
        ),
    )(z, c2, cbm2, cb1, cb2, cb3)

    return zq, codes_t, loss[0, 0]


# speedup vs baseline: 1.1849x; 1.1849x over previous
import jax
import jax.numpy as jnp
from jax.experimental import pallas as pl
from jax.experimental.pallas import tpu as pltpu

_NUM_LEVELS = 4
_N = 1024
_D = 256
_B = 16384
_R = 256
_C = 4
_RB = _R * _C


def _prep_body(cb_ref, c2_ref, cbm2_ref, cb1_ref, cb2_ref, cb3_ref):
    cb = cb_ref[...]
    c2_ref[...] = jnp.sum(cb * cb, axis=-1)
    cb1 = cb.astype(jnp.bfloat16)
    rem = cb - cb1.astype(jnp.float32)
    cb2 = rem.astype(jnp.bfloat16)
    cb3 = (rem - cb2.astype(jnp.float32)).astype(jnp.bfloat16)
    cbm2_ref[...] = jnp.bfloat16(-2.0) * cb1
    cb1_ref[...] = cb1
    cb2_ref[...] = cb2
    cb3_ref[...] = cb3


def _mm_nt(a, b):
    return jax.lax.dot_general(
        a, b, (((1,), (1,)), ((), ())), preferred_element_type=jnp.float32)


def _mm_nn(a, b):
    return jax.lax.dot_general(
        a, b, (((1,), (0,)), ((), ())), preferred_element_type=jnp.float32)


def _vq_body(z_ref, c2_ref, cbm2_ref, cb1_ref, cb2_ref, cb3_ref,
             zq_ref, codes_ref, loss_ref):
    pid = pl.program_id(0)
    iota = jax.lax.broadcasted_iota(jnp.int32, (_R, _N), 1)
    rs = [z_ref[pl.ds(c * _R, _R), :] for c in range(_C)]
    zs = list(rs)
    for lvl in range(_NUM_LEVELS):
        cbm2 = cbm2_ref[lvl]
        cb1 = cb1_ref[lvl]
        cb2 = cb2_ref[lvl]
        cb3 = cb3_ref[lvl]
        c2 = c2_ref[lvl][None, :]
        mms = [_mm_nt(rs[c].astype(jnp.bfloat16), cbm2) for c in range(_C)]
        for c in range(_C):
            r = rs[c]
            r2 = jnp.sum(r * r, axis=1, keepdims=True)
            d = r2 + mms[c] + c2
            dmin = jnp.min(d, axis=1, keepdims=True)
            code = jnp.min(jnp.where(d == dmin, iota, _N), axis=1)
            codes_ref[pl.ds(c * _R, _R), pl.ds(lvl, 1)] = code[:, None]
            onehot = (iota == code[:, None]).astype(jnp.bfloat16)
            sel = (_mm_nn(onehot, cb1) + _mm_nn(onehot, cb2)) + _mm_nn(onehot, cb3)
            rs[c] = r - sel
    part = jnp.zeros((1, 1), jnp.float32)
    for c in range(_C):
        zq_ref[pl.ds(c * _R, _R), :] = zs[c] - rs[c]
        part += jnp.sum(rs[c] * rs[c]).reshape(1, 1)

    @pl.when(pid == 0)
    def _init():
        loss_ref[...] = jnp.zeros_like(loss_ref)

    loss_ref[...] += part * (1.0 / (_B * _D))


@jax.jit
def kernel(z, codebooks):
    cbs_shape = jax.ShapeDtypeStruct((_NUM_LEVELS, _N, _D), jnp.bfloat16)
    c2, cbm2, cb1, cb2, cb3 = pl.pallas_call(
        _prep_body,
        out_shape=[
            jax.ShapeDtypeStruct((_NUM_LEVELS, _N), jnp.float32),
            cbs_shape, cbs_shape, cbs_shape, cbs_shape,
        ],
    )(codebooks)

    grid = _B // _RB
    zq, codes_t, loss = pl.pallas_call(
        _vq_body,
        grid=(grid,),
        in_specs=[
            pl.BlockSpec((_RB, _D), lambda i: (i, 0)),
            pl.BlockSpec((_NUM_LEVELS, _N), lambda i: (0, 0)),
            pl.BlockSpec((_NUM_LEVELS, _N, _D), lambda i: (0, 0, 0)),
            pl.BlockSpec((_NUM_LEVELS, _N, _D), lambda i: (0, 0, 0)),
            pl.BlockSpec((_NUM_LEVELS, _N, _D), lambda i: (0, 0, 0)),
            pl.BlockSpec((_NUM_LEVELS, _N, _D), lambda i: (0, 0, 0)),
        ],
        out_specs=[
            pl.BlockSpec((_RB, _D), lambda i: (i, 0)),
            pl.BlockSpec((_RB, _NUM_LEVELS), lambda i: (i, 0)),
            pl.BlockSpec((1, 1), lambda i: (0, 0)),
        ],
        out_shape=[
            jax.ShapeDtypeStruct((_B, _D), jnp.float32),
            jax.ShapeDtypeStruct((_B, _NUM_LEVELS), jnp.int32),
            jax.ShapeDtypeStruct((1, 1), jnp.float32),
        ],
        compiler_params=pltpu.CompilerParams(
            dimension_semantics=("arbitrary",),
        ),
    )(z, c2, cbm2, cb1, cb2, cb3)

    return zq, codes_t, loss[0, 0]


# speedup vs baseline: 1.2210x; 1.0304x over previous
import jax
import jax.numpy as jnp
from jax.experimental import pallas as pl
from jax.experimental.pallas import tpu as pltpu

_NUM_LEVELS = 4
_N = 1024
_D = 256
_B = 16384
_R = 256
_C = 4
_RB = _R * _C


def _mm_nt(a, b):
    return jax.lax.dot_general(
        a, b, (((1,), (1,)), ((), ())), preferred_element_type=jnp.float32)


def _mm_nn(a, b):
    return jax.lax.dot_general(
        a, b, (((1,), (0,)), ((), ())), preferred_element_type=jnp.float32)


def _vq_body(z_ref, cb_ref, zq_ref, codes_ref, loss_ref,
             c2_s, cbm2_s, cb1_s, cb2_s, cb3_s):
    pid = pl.program_id(0)

    @pl.when(pid == 0)
    def _prep():
        cb = cb_ref[...]
        c2_s[...] = jnp.sum(cb * cb, axis=-1)
        cb1 = cb.astype(jnp.bfloat16)
        rem = cb - cb1.astype(jnp.float32)
        cb2 = rem.astype(jnp.bfloat16)
        cb3 = (rem - cb2.astype(jnp.float32)).astype(jnp.bfloat16)
        cbm2_s[...] = jnp.bfloat16(-2.0) * cb1
        cb1_s[...] = cb1
        cb2_s[...] = cb2
        cb3_s[...] = cb3
        loss_ref[...] = jnp.zeros_like(loss_ref)

    iota = jax.lax.broadcasted_iota(jnp.int32, (_R, _N), 1)
    rs = [z_ref[pl.ds(c * _R, _R), :] for c in range(_C)]
    zs = list(rs)
    for lvl in range(_NUM_LEVELS):
        cbm2 = cbm2_s[lvl]
        cb1 = cb1_s[lvl]
        cb2 = cb2_s[lvl]
        cb3 = cb3_s[lvl]
        c2 = c2_s[lvl][None, :]
        mms = [_mm_nt(rs[c].astype(jnp.bfloat16), cbm2) for c in range(_C)]
        for c in range(_C):
            r = rs[c]
            r2 = jnp.sum(r * r, axis=1, keepdims=True)
            d = r2 + mms[c] + c2
            dmin = jnp.min(d, axis=1, keepdims=True)
            code = jnp.min(jnp.where(d == dmin, iota, _N), axis=1)
            codes_ref[pl.ds(c * _R, _R), pl.ds(lvl, 1)] = code[:, None]
            onehot = (iota == code[:, None]).astype(jnp.bfloat16)
            sel = (_mm_nn(onehot, cb1) + _mm_nn(onehot, cb2)) + _mm_nn(onehot, cb3)
            rs[c] = r - sel
    part = jnp.zeros((1, 1), jnp.float32)
    for c in range(_C):
        zq_ref[pl.ds(c * _R, _R), :] = zs[c] - rs[c]
        part += jnp.sum(rs[c] * rs[c]).reshape(1, 1)

    loss_ref[...] += part * (1.0 / (_B * _D))


@jax.jit
def kernel(z, codebooks):
    grid = _B // _RB
    cbs_scr = pltpu.VMEM((_NUM_LEVELS, _N, _D), jnp.bfloat16)
    zq, codes, loss = pl.pallas_call(
        _vq_body,
        grid=(grid,),
        in_specs=[
            pl.BlockSpec((_RB, _D), lambda i: (i, 0)),
            pl.BlockSpec((_NUM_LEVELS, _N, _D), lambda i: (0, 0, 0)),
        ],
        out_specs=[
            pl.BlockSpec((_RB, _D), lambda i: (i, 0)),
            pl.BlockSpec((_RB, _NUM_LEVELS), lambda i: (i, 0)),
            pl.BlockSpec((1, 1), lambda i: (0, 0)),
        ],
        out_shape=[
            jax.ShapeDtypeStruct((_B, _D), jnp.float32),
            jax.ShapeDtypeStruct((_B, _NUM_LEVELS), jnp.int32),
            jax.ShapeDtypeStruct((1, 1), jnp.float32),
        ],
        scratch_shapes=[
            pltpu.VMEM((_NUM_LEVELS, _N), jnp.float32),
            cbs_scr, cbs_scr, cbs_scr, cbs_scr,
        ],
        compiler_params=pltpu.CompilerParams(
            dimension_semantics=("arbitrary",),
        ),
    )(z, codebooks)

    return zq, codes, loss[0, 0]
